# baseline (speedup 1.0000x reference)
import functools

import jax
import jax.numpy as jnp
from jax import lax
from jax.experimental import pallas as pl
from jax.experimental.pallas import tpu as pltpu
from jax.experimental.pallas import tpu_sc as plsc

BATCH = 16384
DIM = 64
NC = 2
NS = 16
NW = NC * NS
BPW = BATCH // NW
CHUNK = 128
NCHUNK = BPW // CHUNK
L = 16
G = CHUNK // L

_mesh = plsc.VectorSubcoreMesh(core_axis_name="c", subcore_axis_name="s")

_GATHER_DNUMS = lax.GatherDimensionNumbers(
    offset_dims=(), collapsed_slice_dims=(0,), start_index_map=(0,))


def _permute(x, idx):
    return lax.gather(x, idx[:, None], _GATHER_DNUMS, slice_sizes=(1,),
                      mode=lax.GatherScatterMode.PROMISE_IN_BOUNDS)


def _allsum(x, lane):
    for m in (8, 4, 2, 1):
        x = x + _permute(x, lane ^ m)
    return x


@functools.partial(
    pl.kernel,
    mesh=_mesh,
    out_type=jax.ShapeDtypeStruct((BATCH,), jnp.float32),
    compiler_params=pltpu.CompilerParams(needs_layout_passes=False),
    scratch_types=[
        pltpu.VMEM((NCHUNK, CHUNK), jnp.int32),
        pltpu.VMEM((NCHUNK, CHUNK), jnp.int32),
        pltpu.VMEM((NCHUNK, CHUNK), jnp.int32),
        pltpu.VMEM((CHUNK, 2 * DIM), jnp.float32),
        pltpu.VMEM((CHUNK, 2 * DIM), jnp.float32),
        pltpu.VMEM((CHUNK, 2 * DIM), jnp.float32),
        pltpu.VMEM((BPW,), jnp.float32),
        pltpu.SemaphoreType.DMA,
    ],
)
def _complex_sc(hs_hbm, rs_hbm, ts_hbm, ent_hbm, rel_hbm, out_hbm,
                h2, t2, r2, ch, ct, cr, out_v, sem):
    wid = lax.axis_index("s") * NC + lax.axis_index("c")
    base = wid * BPW
    for c in range(NCHUNK):
        off = base + c * CHUNK
        pltpu.sync_copy(hs_hbm.at[pl.ds(off, CHUNK)], h2.at[c])
        pltpu.sync_copy(ts_hbm.at[pl.ds(off, CHUNK)], t2.at[c])
        pltpu.sync_copy(rs_hbm.at[pl.ds(off, CHUNK)], r2.at[c])

    def chunk(c, _):
        copies = [
            pltpu.async_copy(ent_hbm.at[h2.at[c]], ch, sem),
            pltpu.async_copy(ent_hbm.at[t2.at[c]], ct, sem),
            pltpu.async_copy(rel_hbm.at[r2.at[c]], cr, sem),
        ]
        for cp in copies:
            cp.wait()

        def group(g, _):
            lane = lax.iota(jnp.int32, L)
            scores = jnp.zeros((L,), jnp.float32)
            for k in range(L):
                i = g * L + k
                acc = jnp.zeros((L,), jnp.float32)
                for j in range(DIM // L):
                    re_sl = pl.ds(j * L, L)
                    im_sl = pl.ds(DIM + j * L, L)
                    a = ch[i, re_sl]
                    b = ch[i, im_sl]
                    u = ct[i, re_sl]
                    v = ct[i, im_sl]
                    p = cr[i, re_sl]
                    q = cr[i, im_sl]
                    acc = acc + p * (a * u + b * v) + q * (a * v - b * u)
                scores = jnp.where(lane == k, _allsum(acc, lane), scores)
            out_v[pl.ds(c * CHUNK + g * L, L)] = scores
            return 0

        lax.fori_loop(0, G, group, 0)
        return 0

    lax.fori_loop(0, NCHUNK, chunk, 0)
    pltpu.sync_copy(out_v, out_hbm.at[pl.ds(base, BPW)])


_RB = 8000
_NB = 1000000 // _RB


def _concat_body(re_ref, im_ref, out_ref):
    out_ref[:, :DIM] = re_ref[...]
    out_ref[:, DIM:] = im_ref[...]


def _tc_concat(re, im):
    n = re.shape[0]
    rb = _RB if n % _RB == 0 else n
    return pl.pallas_call(
        _concat_body,
        grid=(n // rb,),
        in_specs=[
            pl.BlockSpec((rb, DIM), lambda i: (i, 0)),
            pl.BlockSpec((rb, DIM), lambda i: (i, 0)),
        ],
        out_specs=pl.BlockSpec((rb, 2 * DIM), lambda i: (i, 0)),
        out_shape=jax.ShapeDtypeStruct((n, 2 * DIM), jnp.float32),
    )(re, im)


def kernel(hs, rs, ts, ent_re, ent_im, rel_re, rel_im):
    ent_cat = _tc_concat(ent_re, ent_im)
    rel_cat = _tc_concat(rel_re, rel_im)
    return _complex_sc(hs, rs, ts, ent_cat, rel_cat)


# speedup vs baseline: 1.0053x; 1.0053x over previous
import functools

import jax
import jax.numpy as jnp
from jax import lax
from jax.experimental import pallas as pl
from jax.experimental.pallas import tpu as pltpu
from jax.experimental.pallas import tpu_sc as plsc

BATCH = 16384
DIM = 64
NC = 2
NS = 16
NW = NC * NS
BPW = BATCH // NW
CHUNK = 128
NCHUNK = BPW // CHUNK

_mesh = plsc.VectorSubcoreMesh(core_axis_name="c", subcore_axis_name="s")


@functools.partial(
    pl.kernel,
    mesh=_mesh,
    out_type=(
        jax.ShapeDtypeStruct((2, BATCH, 2 * DIM), jnp.float32),
        jax.ShapeDtypeStruct((BATCH, 2 * DIM), jnp.float32),
    ),
    compiler_params=pltpu.CompilerParams(needs_layout_passes=False),
    scratch_types=[
        pltpu.VMEM((NCHUNK, CHUNK), jnp.int32),
        pltpu.VMEM((NCHUNK, CHUNK), jnp.int32),
        pltpu.VMEM((NCHUNK, CHUNK), jnp.int32),
        pltpu.VMEM((CHUNK, 2 * DIM), jnp.float32),
        pltpu.VMEM((CHUNK, 2 * DIM), jnp.float32),
        pltpu.VMEM((CHUNK, 2 * DIM), jnp.float32),
        pltpu.SemaphoreType.DMA,
    ],
)
def _sc_stage(hs_hbm, rs_hbm, ts_hbm, ent_hbm, rel_hbm,
              pairs_hbm, relrows_hbm,
              h2, t2, r2, bh, bt, br, sem):
    wid = lax.axis_index("s") * NC + lax.axis_index("c")
    base = wid * BPW
    for c in range(NCHUNK):
        off = base + c * CHUNK

        def prep(k, _, c=c, off=off):
            sl = pl.ds(k * 16, 16)
            h2[c, sl] = h2[c, sl] >> 1
            t2[c, sl] = t2[c, sl] >> 1
            return 0

        pltpu.sync_copy(hs_hbm.at[pl.ds(off, CHUNK)], h2.at[c])
        pltpu.sync_copy(ts_hbm.at[pl.ds(off, CHUNK)], t2.at[c])
        pltpu.sync_copy(rs_hbm.at[pl.ds(off, CHUNK)], r2.at[c])
        lax.fori_loop(0, CHUNK // 16, prep, 0)

    def chunk(c, _):
        off = base + c * CHUNK
        copies = [
            pltpu.async_copy(ent_hbm.at[h2.at[c]], bh, sem),
            pltpu.async_copy(ent_hbm.at[t2.at[c]], bt, sem),
            pltpu.async_copy(rel_hbm.at[r2.at[c]], br, sem),
        ]
        for cp in copies:
            cp.wait()
        pltpu.sync_copy(bh, pairs_hbm.at[0, pl.ds(off, CHUNK)])
        pltpu.sync_copy(bt, pairs_hbm.at[1, pl.ds(off, CHUNK)])
        pltpu.sync_copy(br, relrows_hbm.at[pl.ds(off, CHUNK)])
        return 0

    lax.fori_loop(0, NCHUNK, chunk, 0)


_TB = 2048


def _score_body(pre_ref, pim_ref, rel_ref, hs_ref, ts_ref, out_ref):
    s_h = (hs_ref[...] & 1).astype(jnp.float32)
    s_t = (ts_ref[...] & 1).astype(jnp.float32)

    def blend(pair, s):
        return pair[:, :DIM] * (1.0 - s) + pair[:, DIM:] * s

    re_h = blend(pre_ref[0], s_h)
    re_t = blend(pre_ref[1], s_t)
    im_h = blend(pim_ref[0], s_h)
    im_t = blend(pim_ref[1], s_t)
    r_re = rel_ref[:, :DIM]
    r_im = rel_ref[:, DIM:]
    f = r_re * (re_h * re_t + im_h * im_t) + r_im * (re_h * im_t - im_h * re_t)
    out_ref[...] = jnp.sum(f, axis=1, keepdims=True)


def _tc_score(pairs_re, pairs_im, relrows, hs2, ts2):
    return pl.pallas_call(
        _score_body,
        grid=(BATCH // _TB,),
        in_specs=[
            pl.BlockSpec((2, _TB, 2 * DIM), lambda i: (0, i, 0)),
            pl.BlockSpec((2, _TB, 2 * DIM), lambda i: (0, i, 0)),
            pl.BlockSpec((_TB, 2 * DIM), lambda i: (i, 0)),
            pl.BlockSpec((_TB, 1), lambda i: (i, 0)),
            pl.BlockSpec((_TB, 1), lambda i: (i, 0)),
        ],
        out_specs=pl.BlockSpec((_TB, 1), lambda i: (i, 0)),
        out_shape=jax.ShapeDtypeStruct((BATCH, 1), jnp.float32),
    )(pairs_re, pairs_im, relrows, hs2, ts2)


def kernel(hs, rs, ts, ent_re, ent_im, rel_re, rel_im):
    ent_re2 = ent_re.reshape(-1, 2 * DIM)
    ent_im2 = ent_im.reshape(-1, 2 * DIM)
    rel_cat = jnp.concatenate([rel_re, rel_im], axis=1)
    pairs_re, relrows = _sc_stage(hs, rs, ts, ent_re2, rel_cat)
    pairs_im, _ = _sc_stage(hs, rs, ts, ent_im2, rel_cat)
    out = _tc_score(pairs_re, pairs_im, relrows,
                    hs.reshape(-1, 1), ts.reshape(-1, 1))
    return out.reshape(BATCH)
